# Initial kernel scaffold; baseline (speedup 1.0000x reference)
import jax
import jax.numpy as jnp
from jax.experimental import pallas as pl


def kernel(node_h, edge_index, emb_in_w, emb_in_b, edge_w1, edge_b1, edge_w2, edge_b2, coord_w1, coord_b1, coord_w2, node_w1, node_b1, node_w2, node_b2, emb_out_w, emb_out_b, step_count):
    raise NotImplementedError("write your pallas kernel here")


# speedup vs baseline: 2.7368x; 2.7368x over previous
import jax
import jax.numpy as jnp
from jax import lax
from jax.experimental import pallas as pl
from jax.experimental.pallas import tpu as pltpu
from jax.experimental.pallas import tpu_sc as plsc

N = 10000
E = 160000
IN_NF = 8
H = 128
OUT_NF = 4
L = 7

D = 256
DE = 128
N_PAD = 10240

NC = 2
NS = 16
NW = NC * NS
EPW = E // NW
CH = 128
FULL_CHUNKS = EPW // CH
TAIL = EPW - FULL_CHUNKS * CH

ROWS_PER_TILE = N_PAD // NS

_f32 = jnp.float32


def _silu(v):
    return v * jax.nn.sigmoid(v)


def _sc_mesh():
    return plsc.VectorSubcoreMesh(core_axis_name="c", subcore_axis_name="s",
                                  num_cores=NC, num_subcores=NS)


def _gather_body(table, src, dst, out_s, out_d,
                 idx_a, rows_a, idx_b, rows_b, idx_ta, rows_ta,
                 idx_tb, rows_tb, sem_a, sem_b):
    wid = lax.axis_index("s") * NC + lax.axis_index("c")
    base = wid * EPW

    def run(idx_hbm, out_hbm, idx_v, rows_v, idx_t, rows_t, sem):
        def chunk(t, carry):
            off = base + t * CH
            pltpu.sync_copy(idx_hbm.at[pl.ds(off, CH)], idx_v)
            pltpu.async_copy(table.at[idx_v], rows_v, sem).wait()
            pltpu.sync_copy(rows_v, out_hbm.at[pl.ds(off, CH)])
            return carry
        lax.fori_loop(0, FULL_CHUNKS, chunk, 0, unroll=False)
        off = base + FULL_CHUNKS * CH
        pltpu.sync_copy(idx_hbm.at[pl.ds(off, TAIL)], idx_t)
        pltpu.async_copy(table.at[idx_t], rows_t, sem).wait()
        pltpu.sync_copy(rows_t, out_hbm.at[pl.ds(off, TAIL)])

    run(src, out_s, idx_a, rows_a, idx_ta, rows_ta, sem_a)
    run(dst, out_d, idx_b, rows_b, idx_tb, rows_tb, sem_b)


def _sc_gather(table, src, dst):
    return pl.kernel(
        _gather_body,
        out_type=[jax.ShapeDtypeStruct((E, D), _f32),
                  jax.ShapeDtypeStruct((E, D), _f32)],
        mesh=_sc_mesh(),
        scratch_types=[
            pltpu.VMEM((CH,), jnp.int32),
            pltpu.VMEM((CH, D), _f32),
            pltpu.VMEM((CH,), jnp.int32),
            pltpu.VMEM((CH, D), _f32),
            pltpu.VMEM((TAIL,), jnp.int32),
            pltpu.VMEM((TAIL, D), _f32),
            pltpu.VMEM((TAIL,), jnp.int32),
            pltpu.VMEM((TAIL, D), _f32),
            pltpu.SemaphoreType.DMA,
            pltpu.SemaphoreType.DMA,
        ],
    )(table, src, dst)


def _scatter_body(ef, tr, dst, zeros, out_ef, out_tr,
                  idx_v, rows_v, idx_t, rows_t, acc):
    cid = lax.axis_index("c")
    sid = lax.axis_index("s")
    wid = sid * NC + cid
    base = wid * EPW
    my_rows = pl.ds(sid * ROWS_PER_TILE, ROWS_PER_TILE)

    def phase(src_hbm, out_hbm):
        pltpu.sync_copy(zeros.at[my_rows], acc.at[my_rows])
        plsc.subcore_barrier()

        def chunk(t, carry):
            off = base + t * CH
            pltpu.sync_copy(dst.at[pl.ds(off, CH)], idx_v)
            pltpu.sync_copy(src_hbm.at[pl.ds(off, CH)], rows_v)
            pltpu.sync_copy(rows_v, acc.at[idx_v], add=True)
            return carry
        lax.fori_loop(0, FULL_CHUNKS, chunk, 0, unroll=False)
        off = base + FULL_CHUNKS * CH
        pltpu.sync_copy(dst.at[pl.ds(off, TAIL)], idx_t)
        pltpu.sync_copy(src_hbm.at[pl.ds(off, TAIL)], rows_t)
        pltpu.sync_copy(rows_t, acc.at[idx_t], add=True)

        plsc.subcore_barrier()
        pltpu.sync_copy(acc.at[my_rows], out_hbm.at[cid, my_rows])
        plsc.subcore_barrier()

    phase(ef, out_ef)
    phase(tr, out_tr)


def _sc_scatter(ef, tr, dst, zeros):
    return pl.kernel(
        _scatter_body,
        out_type=[jax.ShapeDtypeStruct((NC, N_PAD, DE), _f32),
                  jax.ShapeDtypeStruct((NC, N_PAD, DE), _f32)],
        mesh=_sc_mesh(),
        scratch_types=[
            pltpu.VMEM((CH,), jnp.int32),
            pltpu.VMEM((CH, DE), _f32),
            pltpu.VMEM((TAIL,), jnp.int32),
            pltpu.VMEM((TAIL, DE), _f32),
            pltpu.VMEM_SHARED((N_PAD, DE), _f32),
        ],
    )(ef, tr, dst, zeros)


BE = 1280
BN = 1024


def _edge_tc_body(s_ref, d_ref, w1s_ref, w1d_ref, w1r_ref, b1_ref,
                  w2_ref, b2_ref, wc1_ref, bc1_ref, wc2_ref,
                  ef_ref, tr_ref):
    s = s_ref[...]
    d = d_ref[...]
    hs = s[:, :H]
    hd = d[:, :H]
    diff = s[:, H:H + 4] - d[:, H:H + 4]
    radial = jnp.sum(diff * diff, axis=1, keepdims=True)
    pre1 = (jnp.dot(hs, w1s_ref[...], preferred_element_type=_f32)
            + jnp.dot(hd, w1d_ref[...], preferred_element_type=_f32)
            + radial * w1r_ref[...] + b1_ref[...])
    h1 = _silu(pre1)
    ef = _silu(jnp.dot(h1, w2_ref[...], preferred_element_type=_f32)
               + b2_ref[...])
    g = _silu(jnp.dot(ef, wc1_ref[...], preferred_element_type=_f32)
              + bc1_ref[...])
    scal = jnp.dot(g, wc2_ref[...], preferred_element_type=_f32)
    trans = jnp.clip(diff * scal, -1000.0, 1000.0)
    ones = jnp.ones((s.shape[0], 1), _f32)
    pad = jnp.zeros((s.shape[0], DE - 4), _f32)
    ef_ref[...] = ef
    tr_ref[...] = jnp.concatenate([trans[:, :3], ones, pad], axis=1)


def _edge_tc(srows, drows, w1s, w1d, w1r, b1, w2, b2, wc1, bc1, wc2):
    full = lambda shape: pl.BlockSpec(shape, lambda i: (0, 0))
    return pl.pallas_call(
        _edge_tc_body,
        grid=(E // BE,),
        in_specs=[
            pl.BlockSpec((BE, D), lambda i: (i, 0)),
            pl.BlockSpec((BE, D), lambda i: (i, 0)),
            full((H, H)), full((H, H)), full((1, H)), full((1, H)),
            full((H, H)), full((1, H)),
            full((H, H)), full((1, H)), full((H, 1)),
        ],
        out_specs=[pl.BlockSpec((BE, DE), lambda i: (i, 0)),
                   pl.BlockSpec((BE, DE), lambda i: (i, 0))],
        out_shape=[jax.ShapeDtypeStruct((E, DE), _f32),
                   jax.ShapeDtypeStruct((E, DE), _f32)],
    )(srows, drows, w1s, w1d, w1r, b1, w2, b2, wc1, bc1, wc2)


def _node_tc_body(t_ref, pef_ref, ptr_ref, wa_ref, wb_ref, b1_ref,
                  w2_ref, b2_ref, out_ref):
    t = t_ref[...]
    hh = t[:, :H]
    x4 = t[:, H:H + 4]
    ef_sum = pef_ref[0] + pef_ref[1]
    ptr = ptr_ref[0] + ptr_ref[1]
    tr4 = jnp.concatenate(
        [ptr[:, :3], jnp.zeros((t.shape[0], 1), _f32)], axis=1)
    deg = ptr[:, 3:4]
    denom = jnp.maximum(deg, 1.0)
    xn = jnp.clip(x4, -1000.0, 1000.0) + tr4 / denom
    h1 = _silu(jnp.dot(hh, wa_ref[...], preferred_element_type=_f32)
               + jnp.dot(ef_sum, wb_ref[...], preferred_element_type=_f32)
               + b1_ref[...])
    dh = jnp.dot(h1, w2_ref[...], preferred_element_type=_f32) + b2_ref[...]
    hhn = hh + dh
    pad = jnp.zeros((t.shape[0], D - H - 4), _f32)
    out_ref[...] = jnp.concatenate([hhn, xn, pad], axis=1)


def _node_tc(table, p_ef, p_tr, wa, wb, b1, w2, b2):
    full = lambda shape: pl.BlockSpec(shape, lambda i: (0, 0))
    return pl.pallas_call(
        _node_tc_body,
        grid=(N_PAD // BN,),
        in_specs=[
            pl.BlockSpec((BN, D), lambda i: (i, 0)),
            pl.BlockSpec((NC, BN, DE), lambda i: (0, i, 0)),
            pl.BlockSpec((NC, BN, DE), lambda i: (0, i, 0)),
            full((H, H)), full((H, H)), full((1, H)),
            full((H, H)), full((1, H)),
        ],
        out_specs=pl.BlockSpec((BN, D), lambda i: (i, 0)),
        out_shape=jax.ShapeDtypeStruct((N_PAD, D), _f32),
    )(table, p_ef, p_tr, wa, wb, b1, w2, b2)


def _prologue_body(nh_ref, w_ref, b_ref, out_ref):
    nh = nh_ref[...]
    x = nh[:, 0:3] / 3330.0
    hh = jnp.dot(nh[:, 3:3 + IN_NF], w_ref[...],
                 preferred_element_type=_f32) + b_ref[...]
    pad = jnp.zeros((nh.shape[0], D - H - 3), _f32)
    out_ref[...] = jnp.concatenate([hh, x, pad], axis=1)


def _prologue(node_h_pad, w, b):
    full = lambda shape: pl.BlockSpec(shape, lambda i: (0, 0))
    return pl.pallas_call(
        _prologue_body,
        grid=(N_PAD // BN,),
        in_specs=[
            pl.BlockSpec((BN, 3 + IN_NF), lambda i: (i, 0)),
            full((IN_NF, H)), full((1, H)),
        ],
        out_specs=pl.BlockSpec((BN, D), lambda i: (i, 0)),
        out_shape=jax.ShapeDtypeStruct((N_PAD, D), _f32),
    )(node_h_pad, w, b)


def _epilogue_body(t_ref, wh_ref, wx_ref, b_ref, out_ref):
    t = t_ref[...]
    hh = t[:, :H]
    x3 = t[:, H:H + 3]
    out_ref[...] = (jnp.dot(hh, wh_ref[...], preferred_element_type=_f32)
                    + jnp.dot(x3, wx_ref[...], preferred_element_type=_f32)
                    + b_ref[...])


def _epilogue(table, wh, wx, b):
    full = lambda shape: pl.BlockSpec(shape, lambda i: (0, 0))
    BNo = 1000
    return pl.pallas_call(
        _epilogue_body,
        grid=(N // BNo,),
        in_specs=[
            pl.BlockSpec((BNo, D), lambda i: (i, 0)),
            full((H, OUT_NF)), full((3, OUT_NF)), full((1, OUT_NF)),
        ],
        out_specs=pl.BlockSpec((BNo, OUT_NF), lambda i: (i, 0)),
        out_shape=jax.ShapeDtypeStruct((N, OUT_NF), _f32),
    )(table, wh, wx, b)


def kernel(node_h, edge_index, emb_in_w, emb_in_b, edge_w1, edge_b1,
           edge_w2, edge_b2, coord_w1, coord_b1, coord_w2, node_w1,
           node_b1, node_w2, node_b2, emb_out_w, emb_out_b, step_count):
    src = edge_index[0]
    dst = edge_index[1]
    node_h_pad = jnp.pad(node_h, ((0, N_PAD - N), (0, 0)))
    table = _prologue(node_h_pad, emb_in_w, emb_in_b.reshape(1, H))
    zeros_pad = jnp.zeros((N_PAD, DE), _f32)
    for i in range(L):
        srows, drows = _sc_gather(table, src, dst)
        ef, tr = _edge_tc(srows, drows,
                          edge_w1[i, 1:1 + H], edge_w1[i, 1 + H:1 + 2 * H],
                          edge_w1[i, 0:1], edge_b1[i].reshape(1, H),
                          edge_w2[i], edge_b2[i].reshape(1, H),
                          coord_w1[i], coord_b1[i].reshape(1, H),
                          coord_w2[i])
        p_ef, p_tr = _sc_scatter(ef, tr, dst, zeros_pad)
        table = _node_tc(table, p_ef, p_tr,
                         node_w1[i, :H], node_w1[i, H:],
                         node_b1[i].reshape(1, H),
                         node_w2[i], node_b2[i].reshape(1, H))
    return _epilogue(table, emb_out_w[:H], emb_out_w[H:],
                     emb_out_b.reshape(1, OUT_NF))


# speedup vs baseline: 3.6527x; 1.3346x over previous
import jax
import jax.numpy as jnp
from jax import lax
from jax.experimental import pallas as pl
from jax.experimental.pallas import tpu as pltpu
from jax.experimental.pallas import tpu_sc as plsc

N = 10000
E = 160000
IN_NF = 8
H = 128
OUT_NF = 4
L = 7

D = 256
DE = 128
N_PAD = 10240

NC = 2
NS = 16
EPT = E // NS
CH = 128
FULL_CHUNKS = EPT // CH
TAIL = EPT - FULL_CHUNKS * CH

ROWS_PER_TILE = N_PAD // NS

_f32 = jnp.float32


def _silu(v):
    return v * jax.nn.sigmoid(v)


def _sc_mesh():
    return plsc.VectorSubcoreMesh(core_axis_name="c", subcore_axis_name="s",
                                  num_cores=NC, num_subcores=NS)


def _gather_body(table, eidx_flat, out,
                 idx0, rows0, idx1, rows1, idx_t, rows_t,
                 gsem0, gsem1, wsem0, wsem1, tsem):
    cid = lax.axis_index("c")
    sid = lax.axis_index("s")
    base = sid * EPT
    ibase = cid * E + base
    idx_v = (idx0, idx1)
    rows_v = (rows0, rows1)
    gsem = (gsem0, gsem1)
    wsem = (wsem0, wsem1)

    def load_and_start(t, b):
        pltpu.sync_copy(eidx_flat.at[pl.ds(ibase + t * CH, CH)], idx_v[b])
        pltpu.async_copy(table.at[idx_v[b]], rows_v[b], gsem[b])

    def finish(t, b):
        pltpu.make_async_copy(table.at[idx_v[b]], rows_v[b], gsem[b]).wait()
        off = base + t * CH
        pltpu.async_copy(rows_v[b], out.at[cid, pl.ds(off, CH)], wsem[b])

    def wb_wait(t, b):
        off = base + t * CH
        pltpu.make_async_copy(rows_v[b], out.at[cid, pl.ds(off, CH)],
                              wsem[b]).wait()

    load_and_start(0, 0)
    load_and_start(1, 1)

    def step(i, carry):
        t = i * 2
        finish(t, 0)
        wb_wait(t, 0)
        load_and_start(t + 2, 0)
        finish(t + 1, 1)
        wb_wait(t + 1, 1)
        load_and_start(t + 3, 1)
        return carry
    lax.fori_loop(0, FULL_CHUNKS // 2 - 1, step, 0, unroll=False)

    t = FULL_CHUNKS - 2
    finish(t, 0)
    wb_wait(t, 0)
    finish(t + 1, 1)
    wb_wait(t + 1, 1)

    off = base + FULL_CHUNKS * CH
    pltpu.sync_copy(eidx_flat.at[pl.ds(ibase + FULL_CHUNKS * CH, TAIL)],
                    idx_t)
    pltpu.async_copy(table.at[idx_t], rows_t, tsem).wait()
    pltpu.sync_copy(rows_t, out.at[cid, pl.ds(off, TAIL)])


def _sc_gather(table, eidx_flat):
    return pl.kernel(
        _gather_body,
        out_type=jax.ShapeDtypeStruct((NC, E, D), _f32),
        mesh=_sc_mesh(),
        scratch_types=[
            pltpu.VMEM((CH,), jnp.int32),
            pltpu.VMEM((CH, D), _f32),
            pltpu.VMEM((CH,), jnp.int32),
            pltpu.VMEM((CH, D), _f32),
            pltpu.VMEM((TAIL,), jnp.int32),
            pltpu.VMEM((TAIL, D), _f32),
            pltpu.SemaphoreType.DMA,
            pltpu.SemaphoreType.DMA,
            pltpu.SemaphoreType.DMA,
            pltpu.SemaphoreType.DMA,
            pltpu.SemaphoreType.DMA,
        ],
    )(table, eidx_flat)


def _scatter_body(ef, tr, dst, zeros, out_ef, out_tr,
                  idx0, rows0, idx1, rows1, idx_t, rows_t, acc,
                  lsem0, lsem1):
    cid = lax.axis_index("c")
    sid = lax.axis_index("s")
    base = sid * EPT
    my_rows = pl.ds(sid * ROWS_PER_TILE, ROWS_PER_TILE)
    idx_v = (idx0, idx1)
    rows_v = (rows0, rows1)
    lsem = (lsem0, lsem1)

    pltpu.sync_copy(zeros.at[my_rows], acc.at[my_rows])
    plsc.subcore_barrier()

    def run(src_hbm, out_hbm):
        def load(t, b):
            off = base + t * CH
            pltpu.sync_copy(dst.at[pl.ds(off, CH)], idx_v[b])
            pltpu.async_copy(src_hbm.at[pl.ds(off, CH)], rows_v[b], lsem[b])

        def flush(t, b):
            off = base + t * CH
            pltpu.make_async_copy(src_hbm.at[pl.ds(off, CH)], rows_v[b],
                                  lsem[b]).wait()
            pltpu.sync_copy(rows_v[b], acc.at[idx_v[b]], add=True)

        load(0, 0)
        load(1, 1)

        def step(i, carry):
            t = i * 2
            flush(t, 0)
            load(t + 2, 0)
            flush(t + 1, 1)
            load(t + 3, 1)
            return carry
        lax.fori_loop(0, FULL_CHUNKS // 2 - 1, step, 0, unroll=False)
        t = FULL_CHUNKS - 2
        flush(t, 0)
        flush(t + 1, 1)

        off = base + FULL_CHUNKS * CH
        pltpu.sync_copy(dst.at[pl.ds(off, TAIL)], idx_t)
        pltpu.sync_copy(src_hbm.at[pl.ds(off, TAIL)], rows_t)
        pltpu.sync_copy(rows_t, acc.at[idx_t], add=True)

        plsc.subcore_barrier()
        pltpu.sync_copy(acc.at[my_rows], out_hbm)

    @pl.when(cid == 0)
    def _():
        run(ef, out_ef.at[my_rows])

    @pl.when(cid == 1)
    def _():
        run(tr, out_tr.at[my_rows])


def _sc_scatter(ef, tr, dst, zeros):
    return pl.kernel(
        _scatter_body,
        out_type=[jax.ShapeDtypeStruct((N_PAD, DE), _f32),
                  jax.ShapeDtypeStruct((N_PAD, DE), _f32)],
        mesh=_sc_mesh(),
        scratch_types=[
            pltpu.VMEM((CH,), jnp.int32),
            pltpu.VMEM((CH, DE), _f32),
            pltpu.VMEM((CH,), jnp.int32),
            pltpu.VMEM((CH, DE), _f32),
            pltpu.VMEM((TAIL,), jnp.int32),
            pltpu.VMEM((TAIL, DE), _f32),
            pltpu.VMEM_SHARED((N_PAD, DE), _f32),
            pltpu.SemaphoreType.DMA,
            pltpu.SemaphoreType.DMA,
        ],
    )(ef, tr, dst, zeros)


BE = 1280
BN = 1024


def _edge_tc_body(s_ref, d_ref, w1s_ref, w1d_ref, w1r_ref, b1_ref,
                  w2_ref, b2_ref, wc1_ref, bc1_ref, wc2_ref,
                  ef_ref, tr_ref):
    s = s_ref[0]
    d = d_ref[0]
    hs = s[:, :H]
    hd = d[:, :H]
    diff = s[:, H:H + 4] - d[:, H:H + 4]
    radial = jnp.sum(diff * diff, axis=1, keepdims=True)
    pre1 = (jnp.dot(hs, w1s_ref[...], preferred_element_type=_f32)
            + jnp.dot(hd, w1d_ref[...], preferred_element_type=_f32)
            + radial * w1r_ref[...] + b1_ref[...])
    h1 = _silu(pre1)
    ef = _silu(jnp.dot(h1, w2_ref[...], preferred_element_type=_f32)
               + b2_ref[...])
    g = _silu(jnp.dot(ef, wc1_ref[...], preferred_element_type=_f32)
              + bc1_ref[...])
    scal = jnp.dot(g, wc2_ref[...], preferred_element_type=_f32)
    trans = jnp.clip(diff * scal, -1000.0, 1000.0)
    ones = jnp.ones((s.shape[0], 1), _f32)
    pad = jnp.zeros((s.shape[0], DE - 4), _f32)
    ef_ref[...] = ef
    tr_ref[...] = jnp.concatenate([trans[:, :3], ones, pad], axis=1)


def _edge_tc(rows, w1s, w1d, w1r, b1, w2, b2, wc1, bc1, wc2):
    full = lambda shape: pl.BlockSpec(shape, lambda i: (0, 0))
    return pl.pallas_call(
        _edge_tc_body,
        grid=(E // BE,),
        in_specs=[
            pl.BlockSpec((1, BE, D), lambda i: (0, i, 0)),
            pl.BlockSpec((1, BE, D), lambda i: (1, i, 0)),
            full((H, H)), full((H, H)), full((1, H)), full((1, H)),
            full((H, H)), full((1, H)),
            full((H, H)), full((1, H)), full((H, 1)),
        ],
        out_specs=[pl.BlockSpec((BE, DE), lambda i: (i, 0)),
                   pl.BlockSpec((BE, DE), lambda i: (i, 0))],
        out_shape=[jax.ShapeDtypeStruct((E, DE), _f32),
                   jax.ShapeDtypeStruct((E, DE), _f32)],
    )(rows, rows, w1s, w1d, w1r, b1, w2, b2, wc1, bc1, wc2)


def _node_tc_body(t_ref, pef_ref, ptr_ref, wa_ref, wb_ref, b1_ref,
                  w2_ref, b2_ref, out_ref):
    t = t_ref[...]
    hh = t[:, :H]
    x4 = t[:, H:H + 4]
    ef_sum = pef_ref[...]
    ptr = ptr_ref[...]
    tr4 = jnp.concatenate(
        [ptr[:, :3], jnp.zeros((t.shape[0], 1), _f32)], axis=1)
    deg = ptr[:, 3:4]
    denom = jnp.maximum(deg, 1.0)
    xn = jnp.clip(x4, -1000.0, 1000.0) + tr4 / denom
    h1 = _silu(jnp.dot(hh, wa_ref[...], preferred_element_type=_f32)
               + jnp.dot(ef_sum, wb_ref[...], preferred_element_type=_f32)
               + b1_ref[...])
    dh = jnp.dot(h1, w2_ref[...], preferred_element_type=_f32) + b2_ref[...]
    hhn = hh + dh
    pad = jnp.zeros((t.shape[0], D - H - 4), _f32)
    out_ref[...] = jnp.concatenate([hhn, xn, pad], axis=1)


def _node_tc(table, p_ef, p_tr, wa, wb, b1, w2, b2):
    full = lambda shape: pl.BlockSpec(shape, lambda i: (0, 0))
    return pl.pallas_call(
        _node_tc_body,
        grid=(N_PAD // BN,),
        in_specs=[
            pl.BlockSpec((BN, D), lambda i: (i, 0)),
            pl.BlockSpec((BN, DE), lambda i: (i, 0)),
            pl.BlockSpec((BN, DE), lambda i: (i, 0)),
            full((H, H)), full((H, H)), full((1, H)),
            full((H, H)), full((1, H)),
        ],
        out_specs=pl.BlockSpec((BN, D), lambda i: (i, 0)),
        out_shape=jax.ShapeDtypeStruct((N_PAD, D), _f32),
    )(table, p_ef, p_tr, wa, wb, b1, w2, b2)


def _prologue_body(nh_ref, w_ref, b_ref, out_ref):
    nh = nh_ref[...]
    x = nh[:, 0:3] / 3330.0
    hh = jnp.dot(nh[:, 3:3 + IN_NF], w_ref[...],
                 preferred_element_type=_f32) + b_ref[...]
    pad = jnp.zeros((nh.shape[0], D - H - 3), _f32)
    out_ref[...] = jnp.concatenate([hh, x, pad], axis=1)


def _prologue(node_h_pad, w, b):
    full = lambda shape: pl.BlockSpec(shape, lambda i: (0, 0))
    return pl.pallas_call(
        _prologue_body,
        grid=(N_PAD // BN,),
        in_specs=[
            pl.BlockSpec((BN, 3 + IN_NF), lambda i: (i, 0)),
            full((IN_NF, H)), full((1, H)),
        ],
        out_specs=pl.BlockSpec((BN, D), lambda i: (i, 0)),
        out_shape=jax.ShapeDtypeStruct((N_PAD, D), _f32),
    )(node_h_pad, w, b)


def _epilogue_body(t_ref, wh_ref, wx_ref, b_ref, out_ref):
    t = t_ref[...]
    hh = t[:, :H]
    x3 = t[:, H:H + 3]
    out_ref[...] = (jnp.dot(hh, wh_ref[...], preferred_element_type=_f32)
                    + jnp.dot(x3, wx_ref[...], preferred_element_type=_f32)
                    + b_ref[...])


def _epilogue(table, wh, wx, b):
    full = lambda shape: pl.BlockSpec(shape, lambda i: (0, 0))
    BNo = 1000
    return pl.pallas_call(
        _epilogue_body,
        grid=(N // BNo,),
        in_specs=[
            pl.BlockSpec((BNo, D), lambda i: (i, 0)),
            full((H, OUT_NF)), full((3, OUT_NF)), full((1, OUT_NF)),
        ],
        out_specs=pl.BlockSpec((BNo, OUT_NF), lambda i: (i, 0)),
        out_shape=jax.ShapeDtypeStruct((N, OUT_NF), _f32),
    )(table, wh, wx, b)


def kernel(node_h, edge_index, emb_in_w, emb_in_b, edge_w1, edge_b1,
           edge_w2, edge_b2, coord_w1, coord_b1, coord_w2, node_w1,
           node_b1, node_w2, node_b2, emb_out_w, emb_out_b, step_count):
    dst = edge_index[1]
    eidx_flat = edge_index.reshape(2 * E)
    node_h_pad = jnp.pad(node_h, ((0, N_PAD - N), (0, 0)))
    table = _prologue(node_h_pad, emb_in_w, emb_in_b.reshape(1, H))
    zeros_pad = jnp.zeros((N_PAD, DE), _f32)
    for i in range(L):
        rows = _sc_gather(table, eidx_flat)
        ef, tr = _edge_tc(rows,
                          edge_w1[i, 1:1 + H], edge_w1[i, 1 + H:1 + 2 * H],
                          edge_w1[i, 0:1], edge_b1[i].reshape(1, H),
                          edge_w2[i], edge_b2[i].reshape(1, H),
                          coord_w1[i], coord_b1[i].reshape(1, H),
                          coord_w2[i])
        p_ef, p_tr = _sc_scatter(ef, tr, dst, zeros_pad)
        table = _node_tc(table, p_ef, p_tr,
                         node_w1[i, :H], node_w1[i, H:],
                         node_b1[i].reshape(1, H),
                         node_w2[i], node_b2[i].reshape(1, H))
    return _epilogue(table, emb_out_w[:H], emb_out_w[H:],
                     emb_out_b.reshape(1, OUT_NF))


# speedup vs baseline: 3.8418x; 1.0518x over previous
import jax
import jax.numpy as jnp
from jax import lax
from jax.experimental import pallas as pl
from jax.experimental.pallas import tpu as pltpu
from jax.experimental.pallas import tpu_sc as plsc

N = 10000
E = 160000
IN_NF = 8
H = 128
OUT_NF = 4
L = 7

D = 256
DE = 128
N_PAD = 10240

NC = 2
NS = 16
NH = 2
EH = E // NH
EPT = EH // NS
CH = 128
PIPE_CHUNKS = 38
REM = EPT - PIPE_CHUNKS * CH
TAIL = REM - CH

N_ACC = 10040
ROWS_PER_TILE = 632

_f32 = jnp.float32


def _silu(v):
    return v * jax.nn.sigmoid(v)


def _sc_mesh():
    return plsc.VectorSubcoreMesh(core_axis_name="c", subcore_axis_name="s",
                                  num_cores=NC, num_subcores=NS)


def _make_gather_body(h):
    def _gather_body(table, eidx_flat, out,
                     idx0, rows0, idx1, rows1, idx_r, rows_r, idx_t, rows_t,
                     gsem0, gsem1, wsem0, wsem1, tsem):
        cid = lax.axis_index("c")
        sid = lax.axis_index("s")
        base = sid * EPT
        ibase = cid * EH + base
        idx_v = (idx0, idx1)
        rows_v = (rows0, rows1)
        gsem = (gsem0, gsem1)
        wsem = (wsem0, wsem1)

        def load_and_start(t, b):
            pltpu.sync_copy(eidx_flat.at[pl.ds(ibase + t * CH, CH)],
                            idx_v[b])
            pltpu.async_copy(table.at[idx_v[b]], rows_v[b], gsem[b])

        def finish(t, b):
            pltpu.make_async_copy(table.at[idx_v[b]], rows_v[b],
                                  gsem[b]).wait()
            off = base + t * CH
            pltpu.async_copy(rows_v[b], out.at[cid, pl.ds(off, CH)],
                             wsem[b])

        def wb_wait(t, b):
            off = base + t * CH
            pltpu.make_async_copy(rows_v[b], out.at[cid, pl.ds(off, CH)],
                                  wsem[b]).wait()

        load_and_start(0, 0)
        load_and_start(1, 1)

        def step(i, carry):
            t = i * 2
            finish(t, 0)
            wb_wait(t, 0)
            load_and_start(t + 2, 0)
            finish(t + 1, 1)
            wb_wait(t + 1, 1)
            load_and_start(t + 3, 1)
            return carry
        lax.fori_loop(0, PIPE_CHUNKS // 2 - 1, step, 0, unroll=False)

        t = PIPE_CHUNKS - 2
        finish(t, 0)
        wb_wait(t, 0)
        finish(t + 1, 1)
        wb_wait(t + 1, 1)

        off = base + PIPE_CHUNKS * CH
        pltpu.sync_copy(eidx_flat.at[pl.ds(ibase + PIPE_CHUNKS * CH, CH)],
                        idx_r)
        pltpu.async_copy(table.at[idx_r], rows_r, tsem).wait()
        pltpu.sync_copy(rows_r, out.at[cid, pl.ds(off, CH)])

        off = base + PIPE_CHUNKS * CH + CH
        pltpu.sync_copy(
            eidx_flat.at[pl.ds(ibase + PIPE_CHUNKS * CH + CH, TAIL)], idx_t)
        pltpu.async_copy(table.at[idx_t], rows_t, tsem).wait()
        pltpu.sync_copy(rows_t, out.at[cid, pl.ds(off, TAIL)])
    return _gather_body


def _sc_gather(table, eidx_flat, h):
    return pl.kernel(
        _make_gather_body(h),
        out_type=jax.ShapeDtypeStruct((NC, EH, D), _f32),
        mesh=_sc_mesh(),
        scratch_types=[
            pltpu.VMEM((CH,), jnp.int32),
            pltpu.VMEM((CH, D), _f32),
            pltpu.VMEM((CH,), jnp.int32),
            pltpu.VMEM((CH, D), _f32),
            pltpu.VMEM((CH,), jnp.int32),
            pltpu.VMEM((CH, D), _f32),
            pltpu.VMEM((TAIL,), jnp.int32),
            pltpu.VMEM((TAIL, D), _f32),
            pltpu.SemaphoreType.DMA,
            pltpu.SemaphoreType.DMA,
            pltpu.SemaphoreType.DMA,
            pltpu.SemaphoreType.DMA,
            pltpu.SemaphoreType.DMA,
        ],
    )(table, eidx_flat)


def _scatter_body(ef0, tr0, ef1, tr1, dst, zeros, out_ef, out_tr,
                  idx0, rows0, idx1, rows1, idx_r, rows_r, idx_t, rows_t,
                  acc, lsem0, lsem1):
    cid = lax.axis_index("c")
    sid = lax.axis_index("s")
    base = sid * EPT
    idx_v = (idx0, idx1)
    rows_v = (rows0, rows1)
    lsem = (lsem0, lsem1)

    def stripes(fn):
        @pl.when(sid < NS - 1)
        def _():
            fn(pl.ds(sid * ROWS_PER_TILE, ROWS_PER_TILE))

        @pl.when(sid == NS - 1)
        def _():
            fn(pl.ds((NS - 1) * ROWS_PER_TILE,
                     N_ACC - (NS - 1) * ROWS_PER_TILE))

    stripes(lambda rs: pltpu.sync_copy(zeros.at[rs], acc.at[rs]))
    plsc.subcore_barrier()

    def run_half(src_hbm, dbase):
        def load(t, b):
            off = base + t * CH
            pltpu.sync_copy(dst.at[pl.ds(dbase + off, CH)], idx_v[b])
            pltpu.async_copy(src_hbm.at[pl.ds(off, CH)], rows_v[b], lsem[b])

        def flush(t, b):
            off = base + t * CH
            pltpu.make_async_copy(src_hbm.at[pl.ds(off, CH)], rows_v[b],
                                  lsem[b]).wait()
            pltpu.sync_copy(rows_v[b], acc.at[idx_v[b]], add=True)

        load(0, 0)
        load(1, 1)

        def step(i, carry):
            t = i * 2
            flush(t, 0)
            load(t + 2, 0)
            flush(t + 1, 1)
            load(t + 3, 1)
            return carry
        lax.fori_loop(0, PIPE_CHUNKS // 2 - 1, step, 0, unroll=False)
        t = PIPE_CHUNKS - 2
        flush(t, 0)
        flush(t + 1, 1)

        off = base + PIPE_CHUNKS * CH
        pltpu.sync_copy(dst.at[pl.ds(dbase + off, CH)], idx_r)
        pltpu.sync_copy(src_hbm.at[pl.ds(off, CH)], rows_r)
        pltpu.sync_copy(rows_r, acc.at[idx_r], add=True)

        off = off + CH
        pltpu.sync_copy(dst.at[pl.ds(dbase + off, TAIL)], idx_t)
        pltpu.sync_copy(src_hbm.at[pl.ds(off, TAIL)], rows_t)
        pltpu.sync_copy(rows_t, acc.at[idx_t], add=True)

    def run(a_hbm, b_hbm, out_hbm):
        run_half(a_hbm, 0)
        run_half(b_hbm, EH)
        plsc.subcore_barrier()
        stripes(lambda rs: pltpu.sync_copy(acc.at[rs], out_hbm.at[rs]))

    @pl.when(cid == 0)
    def _():
        run(ef0, ef1, out_ef)

    @pl.when(cid == 1)
    def _():
        run(tr0, tr1, out_tr)


def _sc_scatter(ef0, tr0, ef1, tr1, dst, zeros):
    return pl.kernel(
        _scatter_body,
        out_type=[jax.ShapeDtypeStruct((N_PAD, DE), _f32),
                  jax.ShapeDtypeStruct((N_PAD, DE), _f32)],
        mesh=_sc_mesh(),
        scratch_types=[
            pltpu.VMEM((CH,), jnp.int32),
            pltpu.VMEM((CH, DE), _f32),
            pltpu.VMEM((CH,), jnp.int32),
            pltpu.VMEM((CH, DE), _f32),
            pltpu.VMEM((CH,), jnp.int32),
            pltpu.VMEM((CH, DE), _f32),
            pltpu.VMEM((TAIL,), jnp.int32),
            pltpu.VMEM((TAIL, DE), _f32),
            pltpu.VMEM_SHARED((N_ACC, DE), _f32),
            pltpu.SemaphoreType.DMA,
            pltpu.SemaphoreType.DMA,
        ],
    )(ef0, tr0, ef1, tr1, dst, zeros)


BE = 1600
BN = 1024


def _edge_tc_body(s_ref, d_ref, w1s_ref, w1d_ref, w1r_ref, b1_ref,
                  w2_ref, b2_ref, wc1_ref, bc1_ref, wc2_ref,
                  ef_ref, tr_ref):
    s = s_ref[0]
    d = d_ref[0]
    hs = s[:, :H]
    hd = d[:, :H]
    diff = s[:, H:H + 4] - d[:, H:H + 4]
    radial = jnp.sum(diff * diff, axis=1, keepdims=True)
    pre1 = (jnp.dot(hs, w1s_ref[...], preferred_element_type=_f32)
            + jnp.dot(hd, w1d_ref[...], preferred_element_type=_f32)
            + radial * w1r_ref[...] + b1_ref[...])
    h1 = _silu(pre1)
    ef = _silu(jnp.dot(h1, w2_ref[...], preferred_element_type=_f32)
               + b2_ref[...])
    g = _silu(jnp.dot(ef, wc1_ref[...], preferred_element_type=_f32)
              + bc1_ref[...])
    scal = jnp.dot(g, wc2_ref[...], preferred_element_type=_f32)
    trans = jnp.clip(diff * scal, -1000.0, 1000.0)
    ones = jnp.ones((s.shape[0], 1), _f32)
    pad = jnp.zeros((s.shape[0], DE - 4), _f32)
    ef_ref[...] = ef
    tr_ref[...] = jnp.concatenate([trans[:, :3], ones, pad], axis=1)


def _edge_tc(rows, w1s, w1d, w1r, b1, w2, b2, wc1, bc1, wc2):
    full = lambda shape: pl.BlockSpec(shape, lambda i: (0, 0))
    return pl.pallas_call(
        _edge_tc_body,
        grid=(EH // BE,),
        in_specs=[
            pl.BlockSpec((1, BE, D), lambda i: (0, i, 0)),
            pl.BlockSpec((1, BE, D), lambda i: (1, i, 0)),
            full((H, H)), full((H, H)), full((1, H)), full((1, H)),
            full((H, H)), full((1, H)),
            full((H, H)), full((1, H)), full((H, 1)),
        ],
        out_specs=[pl.BlockSpec((BE, DE), lambda i: (i, 0)),
                   pl.BlockSpec((BE, DE), lambda i: (i, 0))],
        out_shape=[jax.ShapeDtypeStruct((EH, DE), _f32),
                   jax.ShapeDtypeStruct((EH, DE), _f32)],
    )(rows, rows, w1s, w1d, w1r, b1, w2, b2, wc1, bc1, wc2)


def _node_tc_body(t_ref, pef_ref, ptr_ref,
                  wa_ref, wb_ref, b1_ref, w2_ref, b2_ref, out_ref):
    t = t_ref[...]
    hh = t[:, :H]
    x4 = t[:, H:H + 4]
    ef_sum = pef_ref[...]
    ptr = ptr_ref[...]
    tr4 = jnp.concatenate(
        [ptr[:, :3], jnp.zeros((t.shape[0], 1), _f32)], axis=1)
    deg = ptr[:, 3:4]
    denom = jnp.maximum(deg, 1.0)
    xn = jnp.clip(x4, -1000.0, 1000.0) + tr4 / denom
    h1 = _silu(jnp.dot(hh, wa_ref[...], preferred_element_type=_f32)
               + jnp.dot(ef_sum, wb_ref[...], preferred_element_type=_f32)
               + b1_ref[...])
    dh = jnp.dot(h1, w2_ref[...], preferred_element_type=_f32) + b2_ref[...]
    hhn = hh + dh
    pad = jnp.zeros((t.shape[0], D - H - 4), _f32)
    out_ref[...] = jnp.concatenate([hhn, xn, pad], axis=1)


def _node_tc(table, p_ef, p_tr, wa, wb, b1, w2, b2):
    full = lambda shape: pl.BlockSpec(shape, lambda i: (0, 0))
    part = pl.BlockSpec((BN, DE), lambda i: (i, 0))
    return pl.pallas_call(
        _node_tc_body,
        grid=(N_PAD // BN,),
        in_specs=[
            pl.BlockSpec((BN, D), lambda i: (i, 0)),
            part, part,
            full((H, H)), full((H, H)), full((1, H)),
            full((H, H)), full((1, H)),
        ],
        out_specs=pl.BlockSpec((BN, D), lambda i: (i, 0)),
        out_shape=jax.ShapeDtypeStruct((N_PAD, D), _f32),
    )(table, p_ef, p_tr, wa, wb, b1, w2, b2)


def _prologue_body(nh_ref, w_ref, b_ref, out_ref):
    nh = nh_ref[...]
    x = nh[:, 0:3] / 3330.0
    hh = jnp.dot(nh[:, 3:3 + IN_NF], w_ref[...],
                 preferred_element_type=_f32) + b_ref[...]
    pad = jnp.zeros((nh.shape[0], D - H - 3), _f32)
    out_ref[...] = jnp.concatenate([hh, x, pad], axis=1)


def _prologue(node_h_pad, w, b):
    full = lambda shape: pl.BlockSpec(shape, lambda i: (0, 0))
    return pl.pallas_call(
        _prologue_body,
        grid=(N_PAD // BN,),
        in_specs=[
            pl.BlockSpec((BN, 3 + IN_NF), lambda i: (i, 0)),
            full((IN_NF, H)), full((1, H)),
        ],
        out_specs=pl.BlockSpec((BN, D), lambda i: (i, 0)),
        out_shape=jax.ShapeDtypeStruct((N_PAD, D), _f32),
    )(node_h_pad, w, b)


def _epilogue_body(t_ref, wh_ref, wx_ref, b_ref, out_ref):
    t = t_ref[...]
    hh = t[:, :H]
    x3 = t[:, H:H + 3]
    out_ref[...] = (jnp.dot(hh, wh_ref[...], preferred_element_type=_f32)
                    + jnp.dot(x3, wx_ref[...], preferred_element_type=_f32)
                    + b_ref[...])


def _epilogue(table, wh, wx, b):
    full = lambda shape: pl.BlockSpec(shape, lambda i: (0, 0))
    BNo = 1000
    return pl.pallas_call(
        _epilogue_body,
        grid=(N // BNo,),
        in_specs=[
            pl.BlockSpec((BNo, D), lambda i: (i, 0)),
            full((H, OUT_NF)), full((3, OUT_NF)), full((1, OUT_NF)),
        ],
        out_specs=pl.BlockSpec((BNo, OUT_NF), lambda i: (i, 0)),
        out_shape=jax.ShapeDtypeStruct((N, OUT_NF), _f32),
    )(table, wh, wx, b)


def kernel(node_h, edge_index, emb_in_w, emb_in_b, edge_w1, edge_b1,
           edge_w2, edge_b2, coord_w1, coord_b1, coord_w2, node_w1,
           node_b1, node_w2, node_b2, emb_out_w, emb_out_b, step_count):
    dst = edge_index[1]
    eidx_halves = [
        jnp.concatenate([edge_index[0, h * EH:(h + 1) * EH],
                         edge_index[1, h * EH:(h + 1) * EH]])
        for h in range(NH)
    ]
    node_h_pad = jnp.pad(node_h, ((0, N_PAD - N), (0, 0)))
    table = _prologue(node_h_pad, emb_in_w, emb_in_b.reshape(1, H))
    zeros_pad = jnp.zeros((N_PAD, DE), _f32)
    for i in range(L):
        ew = (edge_w1[i, 1:1 + H], edge_w1[i, 1 + H:1 + 2 * H],
              edge_w1[i, 0:1], edge_b1[i].reshape(1, H),
              edge_w2[i], edge_b2[i].reshape(1, H),
              coord_w1[i], coord_b1[i].reshape(1, H), coord_w2[i])
        rows0 = _sc_gather(table, eidx_halves[0], 0)
        ef0, tr0 = _edge_tc(rows0, *ew)
        rows1 = _sc_gather(table, eidx_halves[1], 1)
        ef1, tr1 = _edge_tc(rows1, *ew)
        p_ef, p_tr = _sc_scatter(ef0, tr0, ef1, tr1, dst, zeros_pad)
        table = _node_tc(table, p_ef, p_tr,
                         node_w1[i, :H], node_w1[i, H:],
                         node_b1[i].reshape(1, H),
                         node_w2[i], node_b2[i].reshape(1, H))
    return _epilogue(table, emb_out_w[:H], emb_out_w[H:],
                     emb_out_b.reshape(1, OUT_NF))


# speedup vs baseline: 4.0875x; 1.0640x over previous
import jax
import jax.numpy as jnp
from jax import lax
from jax.experimental import pallas as pl
from jax.experimental.pallas import tpu as pltpu
from jax.experimental.pallas import tpu_sc as plsc

N = 10000
E = 160000
IN_NF = 8
H = 128
OUT_NF = 4
L = 7

D = 256
DE = 128
N_PAD = 10240

NC = 2
NS = 16
NH = 2
EH = E // NH
EPT = EH // NS
CH = 128
PIPE_CHUNKS = 38
REM = EPT - PIPE_CHUNKS * CH
TAIL = REM - CH

N_ACC = 10040
ROWS_PER_TILE = 632

_f32 = jnp.float32


def _silu(v):
    return v * jax.nn.sigmoid(v)


def _sc_mesh():
    return plsc.VectorSubcoreMesh(core_axis_name="c", subcore_axis_name="s",
                                  num_cores=NC, num_subcores=NS)


def _make_gather_body(h):
    def _gather_body(table, eidx_flat, out,
                     idx0, rows0, idx1, rows1, idx_r, rows_r, idx_t, rows_t,
                     gsem0, gsem1, wsem0, wsem1, tsem):
        cid = lax.axis_index("c")
        sid = lax.axis_index("s")
        base = sid * EPT
        ibase = cid * EH + base
        idx_v = (idx0, idx1)
        rows_v = (rows0, rows1)
        gsem = (gsem0, gsem1)
        wsem = (wsem0, wsem1)

        def load_and_start(t, b):
            pltpu.sync_copy(eidx_flat.at[pl.ds(ibase + t * CH, CH)],
                            idx_v[b])
            pltpu.async_copy(table.at[idx_v[b]], rows_v[b], gsem[b])

        def finish(t, b):
            pltpu.make_async_copy(table.at[idx_v[b]], rows_v[b],
                                  gsem[b]).wait()
            off = base + t * CH
            pltpu.async_copy(rows_v[b], out.at[cid, pl.ds(off, CH)],
                             wsem[b])

        def wb_wait(t, b):
            off = base + t * CH
            pltpu.make_async_copy(rows_v[b], out.at[cid, pl.ds(off, CH)],
                                  wsem[b]).wait()

        load_and_start(0, 0)
        load_and_start(1, 1)

        def step(i, carry):
            t = i * 2
            finish(t, 0)
            wb_wait(t, 0)
            load_and_start(t + 2, 0)
            finish(t + 1, 1)
            wb_wait(t + 1, 1)
            load_and_start(t + 3, 1)
            return carry
        lax.fori_loop(0, PIPE_CHUNKS // 2 - 1, step, 0, unroll=False)

        t = PIPE_CHUNKS - 2
        finish(t, 0)
        wb_wait(t, 0)
        finish(t + 1, 1)
        wb_wait(t + 1, 1)

        off = base + PIPE_CHUNKS * CH
        pltpu.sync_copy(eidx_flat.at[pl.ds(ibase + PIPE_CHUNKS * CH, CH)],
                        idx_r)
        pltpu.async_copy(table.at[idx_r], rows_r, tsem).wait()
        pltpu.sync_copy(rows_r, out.at[cid, pl.ds(off, CH)])

        off = base + PIPE_CHUNKS * CH + CH
        pltpu.sync_copy(
            eidx_flat.at[pl.ds(ibase + PIPE_CHUNKS * CH + CH, TAIL)], idx_t)
        pltpu.async_copy(table.at[idx_t], rows_t, tsem).wait()
        pltpu.sync_copy(rows_t, out.at[cid, pl.ds(off, TAIL)])
    return _gather_body


def _sc_gather(table, eidx_flat, h):
    return pl.kernel(
        _make_gather_body(h),
        out_type=jax.ShapeDtypeStruct((NC, EH, D), _f32),
        mesh=_sc_mesh(),
        scratch_types=[
            pltpu.VMEM((CH,), jnp.int32),
            pltpu.VMEM((CH, D), _f32),
            pltpu.VMEM((CH,), jnp.int32),
            pltpu.VMEM((CH, D), _f32),
            pltpu.VMEM((CH,), jnp.int32),
            pltpu.VMEM((CH, D), _f32),
            pltpu.VMEM((TAIL,), jnp.int32),
            pltpu.VMEM((TAIL, D), _f32),
            pltpu.SemaphoreType.DMA,
            pltpu.SemaphoreType.DMA,
            pltpu.SemaphoreType.DMA,
            pltpu.SemaphoreType.DMA,
            pltpu.SemaphoreType.DMA,
        ],
    )(table, eidx_flat)


def _make_scatter_body(dbase):
    def _scatter_body(ef_h, tr_h, dst, init_ef, init_tr, out_ef, out_tr,
                      idx0, rows0, idx1, rows1, idx_r, rows_r, idx_t,
                      rows_t, acc, lsem0, lsem1):
        cid = lax.axis_index("c")
        sid = lax.axis_index("s")
        base = sid * EPT
        idx_v = (idx0, idx1)
        rows_v = (rows0, rows1)
        lsem = (lsem0, lsem1)

        def stripes(fn):
            @pl.when(sid < NS - 1)
            def _():
                fn(pl.ds(sid * ROWS_PER_TILE, ROWS_PER_TILE))

            @pl.when(sid == NS - 1)
            def _():
                fn(pl.ds((NS - 1) * ROWS_PER_TILE,
                         N_ACC - (NS - 1) * ROWS_PER_TILE))

        def run(src_hbm, init_hbm, out_hbm):
            stripes(lambda rs: pltpu.sync_copy(init_hbm.at[rs],
                                               acc.at[rs]))
            plsc.subcore_barrier()

            def load(t, b):
                off = base + t * CH
                pltpu.sync_copy(dst.at[pl.ds(dbase + off, CH)], idx_v[b])
                pltpu.async_copy(src_hbm.at[pl.ds(off, CH)], rows_v[b],
                                 lsem[b])

            def flush(t, b):
                off = base + t * CH
                pltpu.make_async_copy(src_hbm.at[pl.ds(off, CH)],
                                      rows_v[b], lsem[b]).wait()
                pltpu.sync_copy(rows_v[b], acc.at[idx_v[b]], add=True)

            load(0, 0)
            load(1, 1)

            def step(i, carry):
                t = i * 2
                flush(t, 0)
                load(t + 2, 0)
                flush(t + 1, 1)
                load(t + 3, 1)
                return carry
            lax.fori_loop(0, PIPE_CHUNKS // 2 - 1, step, 0, unroll=False)
            t = PIPE_CHUNKS - 2
            flush(t, 0)
            flush(t + 1, 1)

            off = base + PIPE_CHUNKS * CH
            pltpu.sync_copy(dst.at[pl.ds(dbase + off, CH)], idx_r)
            pltpu.sync_copy(src_hbm.at[pl.ds(off, CH)], rows_r)
            pltpu.sync_copy(rows_r, acc.at[idx_r], add=True)

            off = off + CH
            pltpu.sync_copy(dst.at[pl.ds(dbase + off, TAIL)], idx_t)
            pltpu.sync_copy(src_hbm.at[pl.ds(off, TAIL)], rows_t)
            pltpu.sync_copy(rows_t, acc.at[idx_t], add=True)

            plsc.subcore_barrier()
            stripes(lambda rs: pltpu.sync_copy(acc.at[rs],
                                               out_hbm.at[rs]))

        @pl.when(cid == 0)
        def _():
            run(ef_h, init_ef, out_ef)

        @pl.when(cid == 1)
        def _():
            run(tr_h, init_tr, out_tr)
    return _scatter_body


def _sc_scatter(ef_h, tr_h, dst, dbase, init_ef, init_tr):
    return pl.kernel(
        _make_scatter_body(dbase),
        out_type=[jax.ShapeDtypeStruct((N_PAD, DE), _f32),
                  jax.ShapeDtypeStruct((N_PAD, DE), _f32)],
        mesh=_sc_mesh(),
        scratch_types=[
            pltpu.VMEM((CH,), jnp.int32),
            pltpu.VMEM((CH, DE), _f32),
            pltpu.VMEM((CH,), jnp.int32),
            pltpu.VMEM((CH, DE), _f32),
            pltpu.VMEM((CH,), jnp.int32),
            pltpu.VMEM((CH, DE), _f32),
            pltpu.VMEM((TAIL,), jnp.int32),
            pltpu.VMEM((TAIL, DE), _f32),
            pltpu.VMEM_SHARED((N_ACC, DE), _f32),
            pltpu.SemaphoreType.DMA,
            pltpu.SemaphoreType.DMA,
        ],
    )(ef_h, tr_h, dst, init_ef, init_tr)


BE = 1600
BN = 1024


def _edge_tc_body(s_ref, d_ref, w1s_ref, w1d_ref, w1r_ref, b1_ref,
                  w2_ref, b2_ref, wc1_ref, bc1_ref, wc2_ref,
                  ef_ref, tr_ref):
    s = s_ref[0]
    d = d_ref[0]
    hs = s[:, :H]
    hd = d[:, :H]
    diff = s[:, H:H + 4] - d[:, H:H + 4]
    radial = jnp.sum(diff * diff, axis=1, keepdims=True)
    pre1 = (jnp.dot(hs, w1s_ref[...], preferred_element_type=_f32)
            + jnp.dot(hd, w1d_ref[...], preferred_element_type=_f32)
            + radial * w1r_ref[...] + b1_ref[...])
    h1 = _silu(pre1)
    ef = _silu(jnp.dot(h1, w2_ref[...], preferred_element_type=_f32)
               + b2_ref[...])
    g = _silu(jnp.dot(ef, wc1_ref[...], preferred_element_type=_f32)
              + bc1_ref[...])
    scal = jnp.dot(g, wc2_ref[...], preferred_element_type=_f32)
    trans = jnp.clip(diff * scal, -1000.0, 1000.0)
    ones = jnp.ones((s.shape[0], 1), _f32)
    pad = jnp.zeros((s.shape[0], DE - 4), _f32)
    ef_ref[...] = ef
    tr_ref[...] = jnp.concatenate([trans[:, :3], ones, pad], axis=1)


def _edge_tc(rows, w1s, w1d, w1r, b1, w2, b2, wc1, bc1, wc2):
    full = lambda shape: pl.BlockSpec(shape, lambda i: (0, 0))
    return pl.pallas_call(
        _edge_tc_body,
        grid=(EH // BE,),
        in_specs=[
            pl.BlockSpec((1, BE, D), lambda i: (0, i, 0)),
            pl.BlockSpec((1, BE, D), lambda i: (1, i, 0)),
            full((H, H)), full((H, H)), full((1, H)), full((1, H)),
            full((H, H)), full((1, H)),
            full((H, H)), full((1, H)), full((H, 1)),
        ],
        out_specs=[pl.BlockSpec((BE, DE), lambda i: (i, 0)),
                   pl.BlockSpec((BE, DE), lambda i: (i, 0))],
        out_shape=[jax.ShapeDtypeStruct((EH, DE), _f32),
                   jax.ShapeDtypeStruct((EH, DE), _f32)],
    )(rows, rows, w1s, w1d, w1r, b1, w2, b2, wc1, bc1, wc2)


def _node_tc_body(t_ref, pef_ref, ptr_ref,
                  wa_ref, wb_ref, b1_ref, w2_ref, b2_ref, out_ref):
    t = t_ref[...]
    hh = t[:, :H]
    x4 = t[:, H:H + 4]
    ef_sum = pef_ref[...]
    ptr = ptr_ref[...]
    tr4 = jnp.concatenate(
        [ptr[:, :3], jnp.zeros((t.shape[0], 1), _f32)], axis=1)
    deg = ptr[:, 3:4]
    denom = jnp.maximum(deg, 1.0)
    xn = jnp.clip(x4, -1000.0, 1000.0) + tr4 / denom
    h1 = _silu(jnp.dot(hh, wa_ref[...], preferred_element_type=_f32)
               + jnp.dot(ef_sum, wb_ref[...], preferred_element_type=_f32)
               + b1_ref[...])
    dh = jnp.dot(h1, w2_ref[...], preferred_element_type=_f32) + b2_ref[...]
    hhn = hh + dh
    pad = jnp.zeros((t.shape[0], D - H - 4), _f32)
    out_ref[...] = jnp.concatenate([hhn, xn, pad], axis=1)


def _node_tc(table, p_ef, p_tr, wa, wb, b1, w2, b2):
    full = lambda shape: pl.BlockSpec(shape, lambda i: (0, 0))
    part = pl.BlockSpec((BN, DE), lambda i: (i, 0))
    return pl.pallas_call(
        _node_tc_body,
        grid=(N_PAD // BN,),
        in_specs=[
            pl.BlockSpec((BN, D), lambda i: (i, 0)),
            part, part,
            full((H, H)), full((H, H)), full((1, H)),
            full((H, H)), full((1, H)),
        ],
        out_specs=pl.BlockSpec((BN, D), lambda i: (i, 0)),
        out_shape=jax.ShapeDtypeStruct((N_PAD, D), _f32),
    )(table, p_ef, p_tr, wa, wb, b1, w2, b2)


def _prologue_body(nh_ref, w_ref, b_ref, out_ref):
    nh = nh_ref[...]
    x = nh[:, 0:3] / 3330.0
    hh = jnp.dot(nh[:, 3:3 + IN_NF], w_ref[...],
                 preferred_element_type=_f32) + b_ref[...]
    pad = jnp.zeros((nh.shape[0], D - H - 3), _f32)
    out_ref[...] = jnp.concatenate([hh, x, pad], axis=1)


def _prologue(node_h_pad, w, b):
    full = lambda shape: pl.BlockSpec(shape, lambda i: (0, 0))
    return pl.pallas_call(
        _prologue_body,
        grid=(N_PAD // BN,),
        in_specs=[
            pl.BlockSpec((BN, 3 + IN_NF), lambda i: (i, 0)),
            full((IN_NF, H)), full((1, H)),
        ],
        out_specs=pl.BlockSpec((BN, D), lambda i: (i, 0)),
        out_shape=jax.ShapeDtypeStruct((N_PAD, D), _f32),
    )(node_h_pad, w, b)


def _epilogue_body(t_ref, wh_ref, wx_ref, b_ref, out_ref):
    t = t_ref[...]
    hh = t[:, :H]
    x3 = t[:, H:H + 3]
    out_ref[...] = (jnp.dot(hh, wh_ref[...], preferred_element_type=_f32)
                    + jnp.dot(x3, wx_ref[...], preferred_element_type=_f32)
                    + b_ref[...])


def _epilogue(table, wh, wx, b):
    full = lambda shape: pl.BlockSpec(shape, lambda i: (0, 0))
    BNo = 1000
    return pl.pallas_call(
        _epilogue_body,
        grid=(N // BNo,),
        in_specs=[
            pl.BlockSpec((BNo, D), lambda i: (i, 0)),
            full((H, OUT_NF)), full((3, OUT_NF)), full((1, OUT_NF)),
        ],
        out_specs=pl.BlockSpec((BNo, OUT_NF), lambda i: (i, 0)),
        out_shape=jax.ShapeDtypeStruct((N, OUT_NF), _f32),
    )(table, wh, wx, b)


def kernel(node_h, edge_index, emb_in_w, emb_in_b, edge_w1, edge_b1,
           edge_w2, edge_b2, coord_w1, coord_b1, coord_w2, node_w1,
           node_b1, node_w2, node_b2, emb_out_w, emb_out_b, step_count):
    dst = edge_index[1]
    eidx_halves = [
        jnp.concatenate([edge_index[0, h * EH:(h + 1) * EH],
                         edge_index[1, h * EH:(h + 1) * EH]])
        for h in range(NH)
    ]
    node_h_pad = jnp.pad(node_h, ((0, N_PAD - N), (0, 0)))
    table = _prologue(node_h_pad, emb_in_w, emb_in_b.reshape(1, H))
    zeros_pad = jnp.zeros((N_PAD, DE), _f32)
    for i in range(L):
        ew = (edge_w1[i, 1:1 + H], edge_w1[i, 1 + H:1 + 2 * H],
              edge_w1[i, 0:1], edge_b1[i].reshape(1, H),
              edge_w2[i], edge_b2[i].reshape(1, H),
              coord_w1[i], coord_b1[i].reshape(1, H), coord_w2[i])
        rows0 = _sc_gather(table, eidx_halves[0], 0)
        ef0, tr0 = _edge_tc(rows0, *ew)
        rows1 = _sc_gather(table, eidx_halves[1], 1)
        ef1, tr1 = _edge_tc(rows1, *ew)
        p_ef0, p_tr0 = _sc_scatter(ef0, tr0, dst, 0, zeros_pad, zeros_pad)
        p_ef, p_tr = _sc_scatter(ef1, tr1, dst, EH, p_ef0, p_tr0)
        table = _node_tc(table, p_ef, p_tr,
                         node_w1[i, :H], node_w1[i, H:],
                         node_b1[i].reshape(1, H),
                         node_w2[i], node_b2[i].reshape(1, H))
    return _epilogue(table, emb_out_w[:H], emb_out_w[H:],
                     emb_out_b.reshape(1, OUT_NF))


# speedup vs baseline: 4.1486x; 1.0149x over previous
import jax
import jax.numpy as jnp
from jax import lax
from jax.experimental import pallas as pl
from jax.experimental.pallas import tpu as pltpu
from jax.experimental.pallas import tpu_sc as plsc

N = 10000
E = 160000
IN_NF = 8
H = 128
OUT_NF = 4
L = 7

D = 256
DE = 128
N_PAD = 10240

NC = 2
NS = 16
NH = 2
EH = E // NH
EPT = EH // NS
CH = 128
PIPE_CHUNKS = 38
REM = EPT - PIPE_CHUNKS * CH
TAIL = REM - CH

N_ACC = 10040
ROWS_PER_TILE = 632

_f32 = jnp.float32


def _silu(v):
    return v * jax.nn.sigmoid(v)


def _sc_mesh():
    return plsc.VectorSubcoreMesh(core_axis_name="c", subcore_axis_name="s",
                                  num_cores=NC, num_subcores=NS)


GB = 3
GFULL = EPT // CH


def _make_gather_body(h):
    def _gather_body(table, eidx_flat, out,
                     idx_all, rows0, rows1, rows2, idx_t, rows_t,
                     gsem0, gsem1, gsem2, wsem0, wsem1, wsem2, tsem):
        cid = lax.axis_index("c")
        sid = lax.axis_index("s")
        base = sid * EPT
        ibase = cid * EH + base
        rows_v = (rows0, rows1, rows2)
        gsem = (gsem0, gsem1, gsem2)
        wsem = (wsem0, wsem1, wsem2)

        pltpu.sync_copy(eidx_flat.at[pl.ds(ibase, GFULL * CH)], idx_all)

        def idx_at(t):
            return idx_all.at[pl.ds(t * CH, CH)]

        def start(t, b):
            pltpu.async_copy(table.at[idx_at(t)], rows_v[b], gsem[b])

        def finish(t, b):
            pltpu.make_async_copy(table.at[idx_at(t)], rows_v[b],
                                  gsem[b]).wait()
            off = base + t * CH
            pltpu.async_copy(rows_v[b], out.at[cid, pl.ds(off, CH)],
                             wsem[b])

        def wb_wait(t, b):
            off = base + t * CH
            pltpu.make_async_copy(rows_v[b], out.at[cid, pl.ds(off, CH)],
                                  wsem[b]).wait()

        start(0, 0)
        start(1, 1)
        start(2, 2)

        def step(i, carry):
            t = i * GB
            for b in range(GB):
                finish(t + b, b)
                wb_wait(t + b, b)
                start(t + b + GB, b)
            return carry
        lax.fori_loop(0, GFULL // GB - 1, step, 0, unroll=False)

        t = GFULL - GB
        for b in range(GB):
            finish(t + b, b)
            wb_wait(t + b, b)

        off = base + GFULL * CH
        pltpu.sync_copy(eidx_flat.at[pl.ds(ibase + GFULL * CH, TAIL)],
                        idx_t)
        pltpu.async_copy(table.at[idx_t], rows_t, tsem).wait()
        pltpu.sync_copy(rows_t, out.at[cid, pl.ds(off, TAIL)])
    return _gather_body


def _sc_gather(table, eidx_flat, h):
    return pl.kernel(
        _make_gather_body(h),
        out_type=jax.ShapeDtypeStruct((NC, EH, D), _f32),
        mesh=_sc_mesh(),
        scratch_types=[
            pltpu.VMEM((GFULL * CH,), jnp.int32),
            pltpu.VMEM((CH, D), _f32),
            pltpu.VMEM((CH, D), _f32),
            pltpu.VMEM((CH, D), _f32),
            pltpu.VMEM((TAIL,), jnp.int32),
            pltpu.VMEM((TAIL, D), _f32),
            pltpu.SemaphoreType.DMA,
            pltpu.SemaphoreType.DMA,
            pltpu.SemaphoreType.DMA,
            pltpu.SemaphoreType.DMA,
            pltpu.SemaphoreType.DMA,
            pltpu.SemaphoreType.DMA,
            pltpu.SemaphoreType.DMA,
        ],
    )(table, eidx_flat)


def _make_scatter_body(dbase):
    def _scatter_body(ef_h, tr_h, dst, init_ef, init_tr, out_ef, out_tr,
                      idx0, rows0, idx1, rows1, idx_r, rows_r, idx_t,
                      rows_t, acc, lsem0, lsem1, isem0, isem1):
        cid = lax.axis_index("c")
        sid = lax.axis_index("s")
        base = sid * EPT
        idx_v = (idx0, idx1)
        rows_v = (rows0, rows1)
        lsem = (lsem0, lsem1)
        isem = (isem0, isem1)

        def stripes(fn):
            @pl.when(sid < NS - 1)
            def _():
                fn(pl.ds(sid * ROWS_PER_TILE, ROWS_PER_TILE))

            @pl.when(sid == NS - 1)
            def _():
                fn(pl.ds((NS - 1) * ROWS_PER_TILE,
                         N_ACC - (NS - 1) * ROWS_PER_TILE))

        def run(src_hbm, init_hbm, out_hbm):
            stripes(lambda rs: pltpu.sync_copy(init_hbm.at[rs],
                                               acc.at[rs]))
            plsc.subcore_barrier()

            def load(t, b):
                off = base + t * CH
                pltpu.async_copy(dst.at[pl.ds(dbase + off, CH)], idx_v[b],
                                 isem[b])
                pltpu.async_copy(src_hbm.at[pl.ds(off, CH)], rows_v[b],
                                 lsem[b])

            def flush(t, b):
                off = base + t * CH
                pltpu.make_async_copy(dst.at[pl.ds(dbase + off, CH)],
                                      idx_v[b], isem[b]).wait()
                pltpu.make_async_copy(src_hbm.at[pl.ds(off, CH)],
                                      rows_v[b], lsem[b]).wait()
                pltpu.sync_copy(rows_v[b], acc.at[idx_v[b]], add=True)

            load(0, 0)
            load(1, 1)

            def step(i, carry):
                t = i * 2
                flush(t, 0)
                load(t + 2, 0)
                flush(t + 1, 1)
                load(t + 3, 1)
                return carry
            lax.fori_loop(0, PIPE_CHUNKS // 2 - 1, step, 0, unroll=False)
            t = PIPE_CHUNKS - 2
            flush(t, 0)
            flush(t + 1, 1)

            off = base + PIPE_CHUNKS * CH
            pltpu.sync_copy(dst.at[pl.ds(dbase + off, CH)], idx_r)
            pltpu.sync_copy(src_hbm.at[pl.ds(off, CH)], rows_r)
            pltpu.sync_copy(rows_r, acc.at[idx_r], add=True)

            off = off + CH
            pltpu.sync_copy(dst.at[pl.ds(dbase + off, TAIL)], idx_t)
            pltpu.sync_copy(src_hbm.at[pl.ds(off, TAIL)], rows_t)
            pltpu.sync_copy(rows_t, acc.at[idx_t], add=True)

            plsc.subcore_barrier()
            stripes(lambda rs: pltpu.sync_copy(acc.at[rs],
                                               out_hbm.at[rs]))

        @pl.when(cid == 0)
        def _():
            run(ef_h, init_ef, out_ef)

        @pl.when(cid == 1)
        def _():
            run(tr_h, init_tr, out_tr)
    return _scatter_body


def _sc_scatter(ef_h, tr_h, dst, dbase, init_ef, init_tr):
    return pl.kernel(
        _make_scatter_body(dbase),
        out_type=[jax.ShapeDtypeStruct((N_PAD, DE), _f32),
                  jax.ShapeDtypeStruct((N_PAD, DE), _f32)],
        mesh=_sc_mesh(),
        scratch_types=[
            pltpu.VMEM((CH,), jnp.int32),
            pltpu.VMEM((CH, DE), _f32),
            pltpu.VMEM((CH,), jnp.int32),
            pltpu.VMEM((CH, DE), _f32),
            pltpu.VMEM((CH,), jnp.int32),
            pltpu.VMEM((CH, DE), _f32),
            pltpu.VMEM((TAIL,), jnp.int32),
            pltpu.VMEM((TAIL, DE), _f32),
            pltpu.VMEM_SHARED((N_ACC, DE), _f32),
            pltpu.SemaphoreType.DMA,
            pltpu.SemaphoreType.DMA,
            pltpu.SemaphoreType.DMA,
            pltpu.SemaphoreType.DMA,
        ],
    )(ef_h, tr_h, dst, init_ef, init_tr)


BE = 1600
BN = 1024


def _edge_tc_body(s_ref, d_ref, w1s_ref, w1d_ref, w1r_ref, b1_ref,
                  w2_ref, b2_ref, wc1_ref, bc1_ref, wc2_ref,
                  ef_ref, tr_ref):
    s = s_ref[0]
    d = d_ref[0]
    hs = s[:, :H]
    hd = d[:, :H]
    diff = s[:, H:H + 4] - d[:, H:H + 4]
    radial = jnp.sum(diff * diff, axis=1, keepdims=True)
    pre1 = (jnp.dot(hs, w1s_ref[...], preferred_element_type=_f32)
            + jnp.dot(hd, w1d_ref[...], preferred_element_type=_f32)
            + radial * w1r_ref[...] + b1_ref[...])
    h1 = _silu(pre1)
    ef = _silu(jnp.dot(h1, w2_ref[...], preferred_element_type=_f32)
               + b2_ref[...])
    g = _silu(jnp.dot(ef, wc1_ref[...], preferred_element_type=_f32)
              + bc1_ref[...])
    scal = jnp.dot(g, wc2_ref[...], preferred_element_type=_f32)
    trans = jnp.clip(diff * scal, -1000.0, 1000.0)
    ones = jnp.ones((s.shape[0], 1), _f32)
    pad = jnp.zeros((s.shape[0], DE - 4), _f32)
    ef_ref[...] = ef
    tr_ref[...] = jnp.concatenate([trans[:, :3], ones, pad], axis=1)


def _edge_tc(rows, w1s, w1d, w1r, b1, w2, b2, wc1, bc1, wc2):
    full = lambda shape: pl.BlockSpec(shape, lambda i: (0, 0))
    return pl.pallas_call(
        _edge_tc_body,
        grid=(EH // BE,),
        in_specs=[
            pl.BlockSpec((1, BE, D), lambda i: (0, i, 0)),
            pl.BlockSpec((1, BE, D), lambda i: (1, i, 0)),
            full((H, H)), full((H, H)), full((1, H)), full((1, H)),
            full((H, H)), full((1, H)),
            full((H, H)), full((1, H)), full((H, 1)),
        ],
        out_specs=[pl.BlockSpec((BE, DE), lambda i: (i, 0)),
                   pl.BlockSpec((BE, DE), lambda i: (i, 0))],
        out_shape=[jax.ShapeDtypeStruct((EH, DE), _f32),
                   jax.ShapeDtypeStruct((EH, DE), _f32)],
    )(rows, rows, w1s, w1d, w1r, b1, w2, b2, wc1, bc1, wc2)


def _node_tc_body(t_ref, pef_ref, ptr_ref,
                  wa_ref, wb_ref, b1_ref, w2_ref, b2_ref, out_ref):
    t = t_ref[...]
    hh = t[:, :H]
    x4 = t[:, H:H + 4]
    ef_sum = pef_ref[...]
    ptr = ptr_ref[...]
    tr4 = jnp.concatenate(
        [ptr[:, :3], jnp.zeros((t.shape[0], 1), _f32)], axis=1)
    deg = ptr[:, 3:4]
    denom = jnp.maximum(deg, 1.0)
    xn = jnp.clip(x4, -1000.0, 1000.0) + tr4 / denom
    h1 = _silu(jnp.dot(hh, wa_ref[...], preferred_element_type=_f32)
               + jnp.dot(ef_sum, wb_ref[...], preferred_element_type=_f32)
               + b1_ref[...])
    dh = jnp.dot(h1, w2_ref[...], preferred_element_type=_f32) + b2_ref[...]
    hhn = hh + dh
    pad = jnp.zeros((t.shape[0], D - H - 4), _f32)
    out_ref[...] = jnp.concatenate([hhn, xn, pad], axis=1)


def _node_tc(table, p_ef, p_tr, wa, wb, b1, w2, b2):
    full = lambda shape: pl.BlockSpec(shape, lambda i: (0, 0))
    part = pl.BlockSpec((BN, DE), lambda i: (i, 0))
    return pl.pallas_call(
        _node_tc_body,
        grid=(N_PAD // BN,),
        in_specs=[
            pl.BlockSpec((BN, D), lambda i: (i, 0)),
            part, part,
            full((H, H)), full((H, H)), full((1, H)),
            full((H, H)), full((1, H)),
        ],
        out_specs=pl.BlockSpec((BN, D), lambda i: (i, 0)),
        out_shape=jax.ShapeDtypeStruct((N_PAD, D), _f32),
    )(table, p_ef, p_tr, wa, wb, b1, w2, b2)


def _prologue_body(nh_ref, w_ref, b_ref, out_ref):
    nh = nh_ref[...]
    x = nh[:, 0:3] / 3330.0
    hh = jnp.dot(nh[:, 3:3 + IN_NF], w_ref[...],
                 preferred_element_type=_f32) + b_ref[...]
    pad = jnp.zeros((nh.shape[0], D - H - 3), _f32)
    out_ref[...] = jnp.concatenate([hh, x, pad], axis=1)


def _prologue(node_h_pad, w, b):
    full = lambda shape: pl.BlockSpec(shape, lambda i: (0, 0))
    return pl.pallas_call(
        _prologue_body,
        grid=(N_PAD // BN,),
        in_specs=[
            pl.BlockSpec((BN, 3 + IN_NF), lambda i: (i, 0)),
            full((IN_NF, H)), full((1, H)),
        ],
        out_specs=pl.BlockSpec((BN, D), lambda i: (i, 0)),
        out_shape=jax.ShapeDtypeStruct((N_PAD, D), _f32),
    )(node_h_pad, w, b)


def _epilogue_body(t_ref, wh_ref, wx_ref, b_ref, out_ref):
    t = t_ref[...]
    hh = t[:, :H]
    x3 = t[:, H:H + 3]
    out_ref[...] = (jnp.dot(hh, wh_ref[...], preferred_element_type=_f32)
                    + jnp.dot(x3, wx_ref[...], preferred_element_type=_f32)
                    + b_ref[...])


def _epilogue(table, wh, wx, b):
    full = lambda shape: pl.BlockSpec(shape, lambda i: (0, 0))
    BNo = 1000
    return pl.pallas_call(
        _epilogue_body,
        grid=(N // BNo,),
        in_specs=[
            pl.BlockSpec((BNo, D), lambda i: (i, 0)),
            full((H, OUT_NF)), full((3, OUT_NF)), full((1, OUT_NF)),
        ],
        out_specs=pl.BlockSpec((BNo, OUT_NF), lambda i: (i, 0)),
        out_shape=jax.ShapeDtypeStruct((N, OUT_NF), _f32),
    )(table, wh, wx, b)


def kernel(node_h, edge_index, emb_in_w, emb_in_b, edge_w1, edge_b1,
           edge_w2, edge_b2, coord_w1, coord_b1, coord_w2, node_w1,
           node_b1, node_w2, node_b2, emb_out_w, emb_out_b, step_count):
    dst = edge_index[1]
    eidx_halves = [
        jnp.concatenate([edge_index[0, h * EH:(h + 1) * EH],
                         edge_index[1, h * EH:(h + 1) * EH]])
        for h in range(NH)
    ]
    node_h_pad = jnp.pad(node_h, ((0, N_PAD - N), (0, 0)))
    table = _prologue(node_h_pad, emb_in_w, emb_in_b.reshape(1, H))
    zeros_pad = jnp.zeros((N_PAD, DE), _f32)
    for i in range(L):
        ew = (edge_w1[i, 1:1 + H], edge_w1[i, 1 + H:1 + 2 * H],
              edge_w1[i, 0:1], edge_b1[i].reshape(1, H),
              edge_w2[i], edge_b2[i].reshape(1, H),
              coord_w1[i], coord_b1[i].reshape(1, H), coord_w2[i])
        rows0 = _sc_gather(table, eidx_halves[0], 0)
        ef0, tr0 = _edge_tc(rows0, *ew)
        rows1 = _sc_gather(table, eidx_halves[1], 1)
        ef1, tr1 = _edge_tc(rows1, *ew)
        p_ef0, p_tr0 = _sc_scatter(ef0, tr0, dst, 0, zeros_pad, zeros_pad)
        p_ef, p_tr = _sc_scatter(ef1, tr1, dst, EH, p_ef0, p_tr0)
        table = _node_tc(table, p_ef, p_tr,
                         node_w1[i, :H], node_w1[i, H:],
                         node_b1[i].reshape(1, H),
                         node_w2[i], node_b2[i].reshape(1, H))
    return _epilogue(table, emb_out_w[:H], emb_out_w[H:],
                     emb_out_b.reshape(1, OUT_NF))


# speedup vs baseline: 5.8304x; 1.4054x over previous
import jax
import jax.numpy as jnp
from jax import lax
from jax.experimental import pallas as pl
from jax.experimental.pallas import tpu as pltpu
from jax.experimental.pallas import tpu_sc as plsc

N = 10000
E = 160000
IN_NF = 8
H = 128
OUT_NF = 4
L = 7

D = 256
DE = 128
N_PAD = 10240

NC = 2
NS = 16
NH = 2
EH = E // NH
EPT = EH // NS
CH = 128
PIPE_CHUNKS = 38
REM = EPT - PIPE_CHUNKS * CH
TAIL = REM - CH

N_ACC = 10040
ROWS_PER_TILE = 632

_f32 = jnp.float32
_bf16 = jnp.bfloat16


def _silu(v):
    return v * jax.nn.sigmoid(v)


def _sc_mesh():
    return plsc.VectorSubcoreMesh(core_axis_name="c", subcore_axis_name="s",
                                  num_cores=NC, num_subcores=NS)


GB = 3
GFULL = EPT // CH


def _make_gather_body(h):
    def _gather_body(table, eidx_flat, out,
                     idx_all, rows0, rows1, rows2, idx_t, rows_t,
                     gsem0, gsem1, gsem2, wsem0, wsem1, wsem2, tsem):
        cid = lax.axis_index("c")
        sid = lax.axis_index("s")
        base = sid * EPT
        ibase = cid * EH + base
        rows_v = (rows0, rows1, rows2)
        gsem = (gsem0, gsem1, gsem2)
        wsem = (wsem0, wsem1, wsem2)

        pltpu.sync_copy(eidx_flat.at[pl.ds(ibase, GFULL * CH)], idx_all)

        def idx_at(t):
            return idx_all.at[pl.ds(t * CH, CH)]

        def start(t, b):
            pltpu.async_copy(table.at[idx_at(t)], rows_v[b], gsem[b])

        def finish(t, b):
            pltpu.make_async_copy(table.at[idx_at(t)], rows_v[b],
                                  gsem[b]).wait()
            off = base + t * CH
            pltpu.async_copy(rows_v[b], out.at[cid, pl.ds(off, CH)],
                             wsem[b])

        def wb_wait(t, b):
            off = base + t * CH
            pltpu.make_async_copy(rows_v[b], out.at[cid, pl.ds(off, CH)],
                                  wsem[b]).wait()

        start(0, 0)
        start(1, 1)
        start(2, 2)

        def step(i, carry):
            t = i * GB
            for b in range(GB):
                finish(t + b, b)
                wb_wait(t + b, b)
                start(t + b + GB, b)
            return carry
        lax.fori_loop(0, GFULL // GB - 1, step, 0, unroll=False)

        t = GFULL - GB
        for b in range(GB):
            finish(t + b, b)
            wb_wait(t + b, b)

        off = base + GFULL * CH
        pltpu.sync_copy(eidx_flat.at[pl.ds(ibase + GFULL * CH, TAIL)],
                        idx_t)
        pltpu.async_copy(table.at[idx_t], rows_t, tsem).wait()
        pltpu.sync_copy(rows_t, out.at[cid, pl.ds(off, TAIL)])
    return _gather_body


def _sc_gather(table, eidx_flat, h):
    return pl.kernel(
        _make_gather_body(h),
        out_type=jax.ShapeDtypeStruct((NC, EH, H), jnp.uint32),
        mesh=_sc_mesh(),
        scratch_types=[
            pltpu.VMEM((GFULL * CH,), jnp.int32),
            pltpu.VMEM((CH, H), jnp.uint32),
            pltpu.VMEM((CH, H), jnp.uint32),
            pltpu.VMEM((CH, H), jnp.uint32),
            pltpu.VMEM((TAIL,), jnp.int32),
            pltpu.VMEM((TAIL, H), jnp.uint32),
            pltpu.SemaphoreType.DMA,
            pltpu.SemaphoreType.DMA,
            pltpu.SemaphoreType.DMA,
            pltpu.SemaphoreType.DMA,
            pltpu.SemaphoreType.DMA,
            pltpu.SemaphoreType.DMA,
            pltpu.SemaphoreType.DMA,
        ],
    )(table, eidx_flat)


def _make_scatter_body(dbase):
    def _scatter_body(ef_h, tr_h, dst, init_ef, init_tr, out_ef, out_tr,
                      idx0, rows0, idx1, rows1, idx_r, rows_r, idx_t,
                      rows_t, acc, lsem0, lsem1, isem0, isem1):
        cid = lax.axis_index("c")
        sid = lax.axis_index("s")
        base = sid * EPT
        idx_v = (idx0, idx1)
        rows_v = (rows0, rows1)
        lsem = (lsem0, lsem1)
        isem = (isem0, isem1)

        def stripes(fn):
            @pl.when(sid < NS - 1)
            def _():
                fn(pl.ds(sid * ROWS_PER_TILE, ROWS_PER_TILE))

            @pl.when(sid == NS - 1)
            def _():
                fn(pl.ds((NS - 1) * ROWS_PER_TILE,
                         N_ACC - (NS - 1) * ROWS_PER_TILE))

        def run(src_hbm, init_hbm, out_hbm):
            stripes(lambda rs: pltpu.sync_copy(init_hbm.at[rs],
                                               acc.at[rs]))
            plsc.subcore_barrier()

            def load(t, b):
                off = base + t * CH
                pltpu.async_copy(dst.at[pl.ds(dbase + off, CH)], idx_v[b],
                                 isem[b])
                pltpu.async_copy(src_hbm.at[pl.ds(off, CH)], rows_v[b],
                                 lsem[b])

            def flush(t, b):
                off = base + t * CH
                pltpu.make_async_copy(dst.at[pl.ds(dbase + off, CH)],
                                      idx_v[b], isem[b]).wait()
                pltpu.make_async_copy(src_hbm.at[pl.ds(off, CH)],
                                      rows_v[b], lsem[b]).wait()
                pltpu.sync_copy(rows_v[b], acc.at[idx_v[b]], add=True)

            load(0, 0)
            load(1, 1)

            def step(i, carry):
                t = i * 2
                flush(t, 0)
                load(t + 2, 0)
                flush(t + 1, 1)
                load(t + 3, 1)
                return carry
            lax.fori_loop(0, PIPE_CHUNKS // 2 - 1, step, 0, unroll=False)
            t = PIPE_CHUNKS - 2
            flush(t, 0)
            flush(t + 1, 1)

            off = base + PIPE_CHUNKS * CH
            pltpu.sync_copy(dst.at[pl.ds(dbase + off, CH)], idx_r)
            pltpu.sync_copy(src_hbm.at[pl.ds(off, CH)], rows_r)
            pltpu.sync_copy(rows_r, acc.at[idx_r], add=True)

            off = off + CH
            pltpu.sync_copy(dst.at[pl.ds(dbase + off, TAIL)], idx_t)
            pltpu.sync_copy(src_hbm.at[pl.ds(off, TAIL)], rows_t)
            pltpu.sync_copy(rows_t, acc.at[idx_t], add=True)

            plsc.subcore_barrier()
            stripes(lambda rs: pltpu.sync_copy(acc.at[rs],
                                               out_hbm.at[rs]))

        @pl.when(cid == 0)
        def _():
            run(ef_h, init_ef, out_ef)

        @pl.when(cid == 1)
        def _():
            run(tr_h, init_tr, out_tr)
    return _scatter_body


def _sc_scatter(ef_h, tr_h, dst, dbase, init_ef, init_tr):
    return pl.kernel(
        _make_scatter_body(dbase),
        out_type=[jax.ShapeDtypeStruct((N_PAD, DE), _f32),
                  jax.ShapeDtypeStruct((N_PAD, DE), _f32)],
        mesh=_sc_mesh(),
        scratch_types=[
            pltpu.VMEM((CH,), jnp.int32),
            pltpu.VMEM((CH, DE), _f32),
            pltpu.VMEM((CH,), jnp.int32),
            pltpu.VMEM((CH, DE), _f32),
            pltpu.VMEM((CH,), jnp.int32),
            pltpu.VMEM((CH, DE), _f32),
            pltpu.VMEM((TAIL,), jnp.int32),
            pltpu.VMEM((TAIL, DE), _f32),
            pltpu.VMEM_SHARED((N_ACC, DE), _f32),
            pltpu.SemaphoreType.DMA,
            pltpu.SemaphoreType.DMA,
            pltpu.SemaphoreType.DMA,
            pltpu.SemaphoreType.DMA,
        ],
    )(ef_h, tr_h, dst, init_ef, init_tr)


BE = 1600
BN = 1024


def _edge_tc_body(s_ref, d_ref, w1s_ref, w1d_ref, w1r_ref, b1_ref,
                  w2_ref, b2_ref, wc1_ref, bc1_ref, wc2_ref,
                  ef_ref, tr_ref):
    def unpack(w):
        lo = lax.convert_element_type(w & 0xFFFF, jnp.uint16)
        hi = lax.convert_element_type(w >> 16, jnp.uint16)
        hh = lax.bitcast_convert_type(lo, _bf16).astype(_f32)
        xp = lax.bitcast_convert_type(hi, _bf16).astype(_f32)
        return hh, xp

    hs, xs = unpack(s_ref[0])
    hd, xd = unpack(d_ref[0])
    diff = xs[:, :4] - xd[:, :4]
    radial = jnp.sum(diff * diff, axis=1, keepdims=True)
    pre1 = (jnp.dot(hs, w1s_ref[...], preferred_element_type=_f32)
            + jnp.dot(hd, w1d_ref[...], preferred_element_type=_f32)
            + radial * w1r_ref[...] + b1_ref[...])
    h1 = _silu(pre1)
    ef = _silu(jnp.dot(h1, w2_ref[...], preferred_element_type=_f32)
               + b2_ref[...])
    g = _silu(jnp.dot(ef, wc1_ref[...], preferred_element_type=_f32)
              + bc1_ref[...])
    scal = jnp.dot(g, wc2_ref[...], preferred_element_type=_f32)
    trans = jnp.clip(diff * scal, -1000.0, 1000.0)
    ones = jnp.ones((hs.shape[0], 1), _f32)
    pad = jnp.zeros((hs.shape[0], DE - 4), _f32)
    ef_ref[...] = ef
    tr_ref[...] = jnp.concatenate([trans[:, :3], ones, pad], axis=1)


def _edge_tc(rows, w1s, w1d, w1r, b1, w2, b2, wc1, bc1, wc2):
    full = lambda shape: pl.BlockSpec(shape, lambda i: (0, 0))
    return pl.pallas_call(
        _edge_tc_body,
        grid=(EH // BE,),
        in_specs=[
            pl.BlockSpec((1, BE, H), lambda i: (0, i, 0)),
            pl.BlockSpec((1, BE, H), lambda i: (1, i, 0)),
            full((H, H)), full((H, H)), full((1, H)), full((1, H)),
            full((H, H)), full((1, H)),
            full((H, H)), full((1, H)), full((H, 1)),
        ],
        out_specs=[pl.BlockSpec((BE, DE), lambda i: (i, 0)),
                   pl.BlockSpec((BE, DE), lambda i: (i, 0))],
        out_shape=[jax.ShapeDtypeStruct((EH, DE), _f32),
                   jax.ShapeDtypeStruct((EH, DE), _f32)],
    )(rows, rows, w1s, w1d, w1r, b1, w2, b2, wc1, bc1, wc2)


def _pack_words(hh, x4, n):
    xpad = jnp.concatenate([x4, jnp.zeros((n, H - 4), _f32)], axis=1)
    lo = lax.bitcast_convert_type(hh.astype(_bf16), jnp.uint16)
    hi = lax.bitcast_convert_type(xpad.astype(_bf16), jnp.uint16)
    return (lax.convert_element_type(lo, jnp.uint32)
            | (lax.convert_element_type(hi, jnp.uint32) << 16))


def _node_tc_body(t_ref, pef_ref, ptr_ref,
                  wa_ref, wb_ref, b1_ref, w2_ref, b2_ref, out_ref, outb_ref):
    t = t_ref[...]
    hh = t[:, :H]
    x4 = t[:, H:H + 4]
    ef_sum = pef_ref[...]
    ptr = ptr_ref[...]
    tr4 = jnp.concatenate(
        [ptr[:, :3], jnp.zeros((t.shape[0], 1), _f32)], axis=1)
    deg = ptr[:, 3:4]
    denom = jnp.maximum(deg, 1.0)
    xn = jnp.clip(x4, -1000.0, 1000.0) + tr4 / denom
    h1 = _silu(jnp.dot(hh, wa_ref[...], preferred_element_type=_f32)
               + jnp.dot(ef_sum, wb_ref[...], preferred_element_type=_f32)
               + b1_ref[...])
    dh = jnp.dot(h1, w2_ref[...], preferred_element_type=_f32) + b2_ref[...]
    hhn = hh + dh
    pad = jnp.zeros((t.shape[0], D - H - 4), _f32)
    out_ref[...] = jnp.concatenate([hhn, xn, pad], axis=1)
    outb_ref[...] = _pack_words(hhn, xn, t.shape[0])


def _node_tc(table, p_ef, p_tr, wa, wb, b1, w2, b2):
    full = lambda shape: pl.BlockSpec(shape, lambda i: (0, 0))
    part = pl.BlockSpec((BN, DE), lambda i: (i, 0))
    return pl.pallas_call(
        _node_tc_body,
        grid=(N_PAD // BN,),
        in_specs=[
            pl.BlockSpec((BN, D), lambda i: (i, 0)),
            part, part,
            full((H, H)), full((H, H)), full((1, H)),
            full((H, H)), full((1, H)),
        ],
        out_specs=[pl.BlockSpec((BN, D), lambda i: (i, 0)),
                   pl.BlockSpec((BN, H), lambda i: (i, 0))],
        out_shape=[jax.ShapeDtypeStruct((N_PAD, D), _f32),
                   jax.ShapeDtypeStruct((N_PAD, H), jnp.uint32)],
    )(table, p_ef, p_tr, wa, wb, b1, w2, b2)


def _prologue_body(nh_ref, w_ref, b_ref, out_ref, outb_ref):
    nh = nh_ref[...]
    x = nh[:, 0:3] / 3330.0
    hh = jnp.dot(nh[:, 3:3 + IN_NF], w_ref[...],
                 preferred_element_type=_f32) + b_ref[...]
    pad = jnp.zeros((nh.shape[0], D - H - 3), _f32)
    out_ref[...] = jnp.concatenate([hh, x, pad], axis=1)
    x4 = jnp.concatenate([x, jnp.zeros((nh.shape[0], 1), _f32)], axis=1)
    outb_ref[...] = _pack_words(hh, x4, nh.shape[0])


def _prologue(node_h_pad, w, b):
    full = lambda shape: pl.BlockSpec(shape, lambda i: (0, 0))
    return pl.pallas_call(
        _prologue_body,
        grid=(N_PAD // BN,),
        in_specs=[
            pl.BlockSpec((BN, 3 + IN_NF), lambda i: (i, 0)),
            full((IN_NF, H)), full((1, H)),
        ],
        out_specs=[pl.BlockSpec((BN, D), lambda i: (i, 0)),
                   pl.BlockSpec((BN, H), lambda i: (i, 0))],
        out_shape=[jax.ShapeDtypeStruct((N_PAD, D), _f32),
                   jax.ShapeDtypeStruct((N_PAD, H), jnp.uint32)],
    )(node_h_pad, w, b)


def _epilogue_body(t_ref, wh_ref, wx_ref, b_ref, out_ref):
    t = t_ref[...]
    hh = t[:, :H]
    x3 = t[:, H:H + 3]
    out_ref[...] = (jnp.dot(hh, wh_ref[...], preferred_element_type=_f32)
                    + jnp.dot(x3, wx_ref[...], preferred_element_type=_f32)
                    + b_ref[...])


def _epilogue(table, wh, wx, b):
    full = lambda shape: pl.BlockSpec(shape, lambda i: (0, 0))
    BNo = 1000
    return pl.pallas_call(
        _epilogue_body,
        grid=(N // BNo,),
        in_specs=[
            pl.BlockSpec((BNo, D), lambda i: (i, 0)),
            full((H, OUT_NF)), full((3, OUT_NF)), full((1, OUT_NF)),
        ],
        out_specs=pl.BlockSpec((BNo, OUT_NF), lambda i: (i, 0)),
        out_shape=jax.ShapeDtypeStruct((N, OUT_NF), _f32),
    )(table, wh, wx, b)


def kernel(node_h, edge_index, emb_in_w, emb_in_b, edge_w1, edge_b1,
           edge_w2, edge_b2, coord_w1, coord_b1, coord_w2, node_w1,
           node_b1, node_w2, node_b2, emb_out_w, emb_out_b, step_count):
    dst = edge_index[1]
    eidx_halves = [
        jnp.concatenate([edge_index[0, h * EH:(h + 1) * EH],
                         edge_index[1, h * EH:(h + 1) * EH]])
        for h in range(NH)
    ]
    node_h_pad = jnp.pad(node_h, ((0, N_PAD - N), (0, 0)))
    table, table_b = _prologue(node_h_pad, emb_in_w, emb_in_b.reshape(1, H))
    zeros_pad = jnp.zeros((N_PAD, DE), _f32)
    for i in range(L):
        ew = (edge_w1[i, 1:1 + H], edge_w1[i, 1 + H:1 + 2 * H],
              edge_w1[i, 0:1], edge_b1[i].reshape(1, H),
              edge_w2[i], edge_b2[i].reshape(1, H),
              coord_w1[i], coord_b1[i].reshape(1, H), coord_w2[i])
        rows0 = _sc_gather(table_b, eidx_halves[0], 0)
        ef0, tr0 = _edge_tc(rows0, *ew)
        rows1 = _sc_gather(table_b, eidx_halves[1], 1)
        ef1, tr1 = _edge_tc(rows1, *ew)
        p_ef0, p_tr0 = _sc_scatter(ef0, tr0, dst, 0, zeros_pad, zeros_pad)
        p_ef, p_tr = _sc_scatter(ef1, tr1, dst, EH, p_ef0, p_tr0)
        table, table_b = _node_tc(table, p_ef, p_tr,
                                  node_w1[i, :H], node_w1[i, H:],
                                  node_b1[i].reshape(1, H),
                                  node_w2[i], node_b2[i].reshape(1, H))
    return _epilogue(table, emb_out_w[:H], emb_out_w[H:],
                     emb_out_b.reshape(1, OUT_NF))
